# baseline (speedup 1.0000x reference)
import functools

import jax
import jax.numpy as jnp
from jax import lax
from jax.experimental import pallas as pl
from jax.experimental.pallas import tpu as pltpu
from jax.experimental.pallas import tpu_sc as plsc

N = 10000
NP = 10240
E = 320000
D = 128
H = 16
C = 40

NC = 2
NS = 16
EPT = E // NS
CHUNK = 1000
NCHUNK = EPT // CHUNK
ROWS = NP // NS

_mesh = plsc.VectorSubcoreMesh(core_axis_name="c", subcore_axis_name="s")
_sc_params = pltpu.CompilerParams(use_tc_tiling_on_sc=False,
                                  needs_layout_passes=False)


def _lane_splat(vec, j):
    return lax.gather(
        vec, jnp.full((16, 1), j, jnp.int32),
        lax.GatherDimensionNumbers(offset_dims=(),
                                   collapsed_slice_dims=(0,),
                                   start_index_map=(0,)),
        (1,), mode=lax.GatherScatterMode.PROMISE_IN_BOUNDS)


@functools.partial(
    pl.kernel,
    out_type=(jax.ShapeDtypeStruct((NP, H), jnp.float32),
              jax.ShapeDtypeStruct((NP, H), jnp.float32),
              jax.ShapeDtypeStruct((NP, H), jnp.float32)),
    mesh=_mesh,
    scratch_types=[
        pltpu.VMEM((CHUNK,), jnp.int32),
        pltpu.VMEM((CHUNK,), jnp.int32),
        pltpu.VMEM((CHUNK,), jnp.int32),
        pltpu.VMEM((CHUNK,), jnp.int32),
        pltpu.VMEM((CHUNK,), jnp.int32),
        pltpu.VMEM((CHUNK,), jnp.int32),
        pltpu.VMEM((CHUNK,), jnp.int32),
        pltpu.VMEM((CHUNK,), jnp.int32),
        pltpu.VMEM((CHUNK + 8,), jnp.float32),
        pltpu.VMEM((CHUNK + 8,), jnp.float32),
        pltpu.VMEM((CHUNK + 8,), jnp.float32),
        pltpu.VMEM((CHUNK + 8,), jnp.float32),
        pltpu.VMEM((CHUNK, H), jnp.float32),
        pltpu.VMEM((CHUNK, H), jnp.float32),
        pltpu.VMEM((ROWS,), jnp.float32),
        pltpu.VMEM((ROWS, H), jnp.float32),
        pltpu.VMEM((ROWS, H), jnp.float32),
        pltpu.VMEM((ROWS, H), jnp.float32),
        pltpu.VMEM((ROWS,), jnp.float32),
        pltpu.VMEM((16,), jnp.float32),
        pltpu.VMEM_SHARED((NP, H), jnp.float32),
        pltpu.VMEM_SHARED((NP,), jnp.float32),
        pltpu.SemaphoreType.DMA,
        pltpu.SemaphoreType.DMA,
        pltpu.SemaphoreType.DMA,
        pltpu.SemaphoreType.DMA,
        pltpu.SemaphoreType.DMA,
        pltpu.SemaphoreType.DMA,
        pltpu.SemaphoreType.DMA,
        pltpu.SemaphoreType.DMA,
        pltpu.SemaphoreType.DMA,
        pltpu.SemaphoreType.DMA,
    ],
    compiler_params=_sc_params,
)
def _sc_mega(h_hbm, src_hbm, dst_hbm, w_hbm, b1_hbm,
             out_hbm, p1_hbm, p2_hbm,
             src_v0, src_v1, src_v2, src_v3,
             dst_v0, dst_v1, dst_v2, dst_v3,
             w_v0, w_v1, w_v2, w_v3,
             rows_v0, rows_v1, zdeg_v, h_v, p_v, a_v,
             dinv_v, b1_v, acc_sp, deg_sp,
             isem0, isem1, isem2, isem3,
             gsem0, gsem1, ssem0, ssem1, ssem2, ssem3):
    c = lax.axis_index("c")
    s = lax.axis_index("s")
    src_v = (src_v0, src_v1, src_v2, src_v3)
    dst_v = (dst_v0, dst_v1, dst_v2, dst_v3)
    w_v = (w_v0, w_v1, w_v2, w_v3)
    rows_v = (rows_v0, rows_v1)
    isem = (isem0, isem1, isem2, isem3)
    gsem = (gsem0, gsem1)
    ssem = (ssem0, ssem1, ssem2, ssem3)
    rbase = s * ROWS

    def z16(i, carry):
        zdeg_v[pl.ds(i * 16, 16)] = jnp.zeros((16,), jnp.float32)
        return carry
    lax.fori_loop(0, ROWS // 16, z16, 0)

    def zrow(i, carry):
        p_v[i, :] = jnp.zeros((H,), jnp.float32)
        return carry
    lax.fori_loop(0, ROWS, zrow, 0)

    pltpu.sync_copy(zdeg_v, deg_sp.at[pl.ds(rbase, ROWS)])
    pltpu.sync_copy(p_v, acc_sp.at[pl.ds(rbase, ROWS)])
    pltpu.sync_copy(h_hbm.at[pl.ds(rbase, ROWS)], h_v)
    pltpu.sync_copy(b1_hbm, b1_v)
    plsc.subcore_barrier()

    def start_idx(k, with_src):
        b = k % 4
        base = s * EPT + k * CHUNK
        if with_src:
            pltpu.async_copy(src_hbm.at[pl.ds(base, CHUNK)], src_v[b],
                             isem[b])
        pltpu.async_copy(dst_hbm.at[pl.ds(base, CHUNK)], dst_v[b], isem[b])
        pltpu.async_copy(w_hbm.at[pl.ds(base, CHUNK)],
                         w_v[b].at[pl.ds(0, CHUNK)], isem[b])

    def wait_idx(k, with_src):
        b = k % 4
        if with_src:
            pltpu.make_async_copy(src_hbm.at[pl.ds(0, CHUNK)], src_v[b],
                                  isem[b]).wait()
        pltpu.make_async_copy(dst_hbm.at[pl.ds(0, CHUNK)], dst_v[b],
                              isem[b]).wait()
        pltpu.make_async_copy(w_hbm.at[pl.ds(0, CHUNK)],
                              w_v[b].at[pl.ds(0, CHUNK)], isem[b]).wait()

    def deg_start_scatter(k):
        b = k % 4
        pltpu.async_copy(w_v[b].at[pl.ds(0, CHUNK)], deg_sp.at[dst_v[b]],
                         ssem[b], add=True)

    def deg_wait_scatter(k):
        b = k % 4
        pltpu.make_async_copy(w_v[b].at[pl.ds(0, CHUNK)],
                              deg_sp.at[dst_v[b]], ssem[b]).wait()

    start_idx(0, False)
    start_idx(1, False)
    for k in range(NCHUNK):
        if k >= 2:
            deg_wait_scatter(k - 2)
        wait_idx(k, False)
        if k + 2 < NCHUNK:
            start_idx(k + 2, False)
        deg_start_scatter(k)
    deg_wait_scatter(NCHUNK - 2)
    deg_wait_scatter(NCHUNK - 1)
    plsc.subcore_barrier()

    pltpu.sync_copy(deg_sp.at[pl.ds(rbase, ROWS)], dinv_v)

    def dinv_body(g, carry):
        x = dinv_v[pl.ds(g * 16, 16)] + 1.0
        i = plsc.bitcast(x, jnp.int32)
        y = plsc.bitcast(jnp.int32(0x5F3759DF) - (i >> 1), jnp.float32)
        for _ in range(3):
            y = y * (1.5 - 0.5 * x * y * y)
        dinv_v[pl.ds(g * 16, 16)] = y
        return carry
    lax.fori_loop(0, ROWS // 16, dinv_body, 0)

    def p1_body(g, carry):
        dv = dinv_v[pl.ds(g * 16, 16)]
        for j in range(16):
            r = g * 16 + j
            p_v[r, :] = h_v[r, :] * _lane_splat(dv, j)
        return carry
    lax.fori_loop(0, ROWS // 16, p1_body, 0)
    pltpu.sync_copy(p_v, p1_hbm.at[pl.ds(rbase, ROWS)])
    plsc.subcore_barrier()

    def scale_rows(k):
        wb = w_v[k % 4]
        rb = rows_v[k % 2]

        def body(g, carry):
            wv = wb[pl.ds(g * 16, 16)]
            for j in range(16):
                e = g * 16 + j
                rb[e, :] = rb[e, :] * _lane_splat(wv, j)
            return carry
        lax.fori_loop(0, CHUNK // 16, body, 0)
        nfull = (CHUNK // 16) * 16
        if CHUNK - nfull:
            wv = wb[pl.ds(nfull, 16)]
            for j in range(CHUNK - nfull):
                rb[nfull + j, :] = rb[nfull + j, :] * _lane_splat(wv, j)

    def run_agg(ptab_hbm):
        def start_gather(k):
            pltpu.async_copy(ptab_hbm.at[src_v[k % 4]], rows_v[k % 2],
                             gsem[k % 2])

        def wait_gather(k):
            pltpu.make_async_copy(ptab_hbm.at[src_v[k % 4]], rows_v[k % 2],
                                  gsem[k % 2]).wait()

        def start_scatter(k):
            pltpu.async_copy(rows_v[k % 2], acc_sp.at[dst_v[k % 4]],
                             ssem[k % 4], add=True)

        def wait_scatter(k):
            pltpu.make_async_copy(rows_v[k % 2], acc_sp.at[dst_v[k % 4]],
                                  ssem[k % 4]).wait()

        start_idx(0, True)
        start_idx(1, True)
        wait_idx(0, True)
        start_gather(0)
        for k in range(NCHUNK):
            wait_gather(k)
            if k >= 1:
                wait_scatter(k - 1)
            if k + 1 < NCHUNK:
                wait_idx(k + 1, True)
                start_gather(k + 1)
            if k + 2 < NCHUNK:
                start_idx(k + 2, True)
            scale_rows(k)
            start_scatter(k)
        wait_scatter(NCHUNK - 1)
        plsc.subcore_barrier()

    run_agg(p1_hbm)

    pltpu.sync_copy(acc_sp.at[pl.ds(rbase, ROWS)], a_v)
    b1row = b1_v[...]

    def mid_body(g, carry):
        dv = dinv_v[pl.ds(g * 16, 16)]
        for j in range(16):
            r = g * 16 + j
            dsp = _lane_splat(dv, j)
            s1 = dsp * a_v[r, :] + dsp * dsp * h_v[r, :] + b1row
            rr = jnp.maximum(s1, 0.0)
            h_v[r, :] = rr
            p_v[r, :] = dsp * rr
            a_v[r, :] = jnp.zeros((H,), jnp.float32)
        return carry
    lax.fori_loop(0, ROWS // 16, mid_body, 0)
    pltpu.sync_copy(p_v, p2_hbm.at[pl.ds(rbase, ROWS)])
    pltpu.sync_copy(a_v, acc_sp.at[pl.ds(rbase, ROWS)])
    plsc.subcore_barrier()

    run_agg(p2_hbm)

    pltpu.sync_copy(acc_sp.at[pl.ds(rbase, ROWS)], a_v)

    def out_body(g, carry):
        dv = dinv_v[pl.ds(g * 16, 16)]
        for j in range(16):
            r = g * 16 + j
            dsp = _lane_splat(dv, j)
            a_v[r, :] = dsp * a_v[r, :] + dsp * dsp * h_v[r, :]
        return carry
    lax.fori_loop(0, ROWS // 16, out_body, 0)

    @pl.when((s < 8) == (c == 0))
    def _():
        pltpu.sync_copy(a_v, out_hbm.at[pl.ds(rbase, ROWS)])


def _tc_h1_body(x_ref, w1_ref, h_ref):
    h = jnp.dot(x_ref[:], w1_ref[:], preferred_element_type=jnp.float32)
    h_ref[0:N, :] = h
    h_ref[N:NP, :] = jnp.zeros((NP - N, H), jnp.float32)


def _tc_out_body(s2_ref, w2_ref, b2_ref, o_ref):
    z = jnp.dot(s2_ref[0:N, :], w2_ref[:],
                preferred_element_type=jnp.float32) + b2_ref[:]
    m = jnp.max(z, axis=1, keepdims=True)
    ez = jnp.exp(z - m)
    lse = jnp.log(jnp.sum(ez, axis=1, keepdims=True)) + m
    o_ref[:] = z - lse


_tc_h1 = pl.pallas_call(
    _tc_h1_body,
    out_shape=jax.ShapeDtypeStruct((NP, H), jnp.float32),
)

_tc_out = pl.pallas_call(
    _tc_out_body,
    out_shape=jax.ShapeDtypeStruct((N, C), jnp.float32),
)


@jax.jit
def kernel(x, edge_index, edge_attr, W1, b1, W2, b2):
    src = edge_index[0]
    dst = edge_index[1]
    h1 = _tc_h1(x, W1)
    s2, _, _ = _sc_mega(h1, src, dst, edge_attr, b1)
    return _tc_out(s2, W2, b2.reshape(1, C))


# speedup vs baseline: 1.0361x; 1.0361x over previous
import functools

import jax
import jax.numpy as jnp
from jax import lax
from jax.experimental import pallas as pl
from jax.experimental.pallas import tpu as pltpu
from jax.experimental.pallas import tpu_sc as plsc

N = 10000
NP = 10240
E = 320000
D = 128
H = 16
C = 40

NC = 2
NS = 16
EPT = E // NS
CHUNK = 1000
NCHUNK = EPT // CHUNK
ROWS = NP // NS

_mesh = plsc.VectorSubcoreMesh(core_axis_name="c", subcore_axis_name="s")
_sc_params = pltpu.CompilerParams(use_tc_tiling_on_sc=False,
                                  needs_layout_passes=False)


def _lane_splat(vec, j):
    return lax.gather(
        vec, jnp.full((16, 1), j, jnp.int32),
        lax.GatherDimensionNumbers(offset_dims=(),
                                   collapsed_slice_dims=(0,),
                                   start_index_map=(0,)),
        (1,), mode=lax.GatherScatterMode.PROMISE_IN_BOUNDS)


@functools.partial(
    pl.kernel,
    out_type=jax.ShapeDtypeStruct((NP, H), jnp.float32),
    mesh=_mesh,
    scratch_types=[
        pltpu.VMEM((CHUNK,), jnp.int32),
        pltpu.VMEM((CHUNK,), jnp.int32),
        pltpu.VMEM((CHUNK,), jnp.int32),
        pltpu.VMEM((CHUNK,), jnp.int32),
        pltpu.VMEM((CHUNK,), jnp.int32),
        pltpu.VMEM((CHUNK,), jnp.int32),
        pltpu.VMEM((CHUNK,), jnp.int32),
        pltpu.VMEM((CHUNK,), jnp.int32),
        pltpu.VMEM((CHUNK + 8,), jnp.float32),
        pltpu.VMEM((CHUNK + 8,), jnp.float32),
        pltpu.VMEM((CHUNK + 8,), jnp.float32),
        pltpu.VMEM((CHUNK + 8,), jnp.float32),
        pltpu.VMEM((CHUNK, H), jnp.float32),
        pltpu.VMEM((CHUNK, H), jnp.float32),
        pltpu.VMEM((ROWS,), jnp.float32),
        pltpu.VMEM((ROWS, H), jnp.float32),
        pltpu.VMEM((ROWS, H), jnp.float32),
        pltpu.VMEM((ROWS, H), jnp.float32),
        pltpu.VMEM((ROWS,), jnp.float32),
        pltpu.VMEM((16,), jnp.float32),
        pltpu.VMEM_SHARED((NP, H), jnp.float32),
        pltpu.VMEM_SHARED((NP, H), jnp.float32),
        pltpu.VMEM_SHARED((NP,), jnp.float32),
        pltpu.SemaphoreType.DMA,
        pltpu.SemaphoreType.DMA,
        pltpu.SemaphoreType.DMA,
        pltpu.SemaphoreType.DMA,
        pltpu.SemaphoreType.DMA,
        pltpu.SemaphoreType.DMA,
        pltpu.SemaphoreType.DMA,
        pltpu.SemaphoreType.DMA,
        pltpu.SemaphoreType.DMA,
        pltpu.SemaphoreType.DMA,
    ],
    compiler_params=_sc_params,
)
def _sc_mega(h_hbm, src_hbm, dst_hbm, w_hbm, b1_hbm, out_hbm,
             src_v0, src_v1, src_v2, src_v3,
             dst_v0, dst_v1, dst_v2, dst_v3,
             w_v0, w_v1, w_v2, w_v3,
             rows_v0, rows_v1, zdeg_v, h_v, p_v, a_v,
             dinv_v, b1_v, p_sp, acc_sp, deg_sp,
             isem0, isem1, isem2, isem3,
             gsem0, gsem1, ssem0, ssem1, ssem2, ssem3):
    c = lax.axis_index("c")
    s = lax.axis_index("s")
    src_v = (src_v0, src_v1, src_v2, src_v3)
    dst_v = (dst_v0, dst_v1, dst_v2, dst_v3)
    w_v = (w_v0, w_v1, w_v2, w_v3)
    rows_v = (rows_v0, rows_v1)
    isem = (isem0, isem1, isem2, isem3)
    gsem = (gsem0, gsem1)
    ssem = (ssem0, ssem1, ssem2, ssem3)
    rbase = s * ROWS

    def z16(i, carry):
        zdeg_v[pl.ds(i * 16, 16)] = jnp.zeros((16,), jnp.float32)
        return carry
    lax.fori_loop(0, ROWS // 16, z16, 0)

    def zrow(i, carry):
        p_v[i, :] = jnp.zeros((H,), jnp.float32)
        return carry
    lax.fori_loop(0, ROWS, zrow, 0)

    pltpu.sync_copy(zdeg_v, deg_sp.at[pl.ds(rbase, ROWS)])
    pltpu.sync_copy(p_v, acc_sp.at[pl.ds(rbase, ROWS)])
    pltpu.sync_copy(h_hbm.at[pl.ds(rbase, ROWS)], h_v)
    pltpu.sync_copy(b1_hbm, b1_v)
    plsc.subcore_barrier()

    def start_idx(k, with_src):
        b = k % 4
        base = s * EPT + k * CHUNK
        if with_src:
            pltpu.async_copy(src_hbm.at[pl.ds(base, CHUNK)], src_v[b],
                             isem[b])
        pltpu.async_copy(dst_hbm.at[pl.ds(base, CHUNK)], dst_v[b], isem[b])
        pltpu.async_copy(w_hbm.at[pl.ds(base, CHUNK)],
                         w_v[b].at[pl.ds(0, CHUNK)], isem[b])

    def wait_idx(k, with_src):
        b = k % 4
        if with_src:
            pltpu.make_async_copy(src_hbm.at[pl.ds(0, CHUNK)], src_v[b],
                                  isem[b]).wait()
        pltpu.make_async_copy(dst_hbm.at[pl.ds(0, CHUNK)], dst_v[b],
                              isem[b]).wait()
        pltpu.make_async_copy(w_hbm.at[pl.ds(0, CHUNK)],
                              w_v[b].at[pl.ds(0, CHUNK)], isem[b]).wait()

    def deg_start_scatter(k):
        b = k % 4
        pltpu.async_copy(w_v[b].at[pl.ds(0, CHUNK)], deg_sp.at[dst_v[b]],
                         ssem[b], add=True)

    def deg_wait_scatter(k):
        b = k % 4
        pltpu.make_async_copy(w_v[b].at[pl.ds(0, CHUNK)],
                              deg_sp.at[dst_v[b]], ssem[b]).wait()

    start_idx(0, False)
    start_idx(1, False)
    for k in range(NCHUNK):
        if k >= 2:
            deg_wait_scatter(k - 2)
        wait_idx(k, False)
        if k + 2 < NCHUNK:
            start_idx(k + 2, False)
        deg_start_scatter(k)
    deg_wait_scatter(NCHUNK - 2)
    deg_wait_scatter(NCHUNK - 1)
    plsc.subcore_barrier()

    pltpu.sync_copy(deg_sp.at[pl.ds(rbase, ROWS)], dinv_v)

    def dinv_body(g, carry):
        x = dinv_v[pl.ds(g * 16, 16)] + 1.0
        i = plsc.bitcast(x, jnp.int32)
        y = plsc.bitcast(jnp.int32(0x5F3759DF) - (i >> 1), jnp.float32)
        for _ in range(3):
            y = y * (1.5 - 0.5 * x * y * y)
        dinv_v[pl.ds(g * 16, 16)] = y
        return carry
    lax.fori_loop(0, ROWS // 16, dinv_body, 0)

    def p1_body(g, carry):
        dv = dinv_v[pl.ds(g * 16, 16)]
        for j in range(16):
            r = g * 16 + j
            p_v[r, :] = h_v[r, :] * _lane_splat(dv, j)
        return carry
    lax.fori_loop(0, ROWS // 16, p1_body, 0)
    pltpu.sync_copy(p_v, p_sp.at[pl.ds(rbase, ROWS)])
    plsc.subcore_barrier()

    def scale_rows(k):
        wb = w_v[k % 4]
        rb = rows_v[k % 2]

        def body(g, carry):
            wv = wb[pl.ds(g * 16, 16)]
            for j in range(16):
                e = g * 16 + j
                rb[e, :] = rb[e, :] * _lane_splat(wv, j)
            return carry
        lax.fori_loop(0, CHUNK // 16, body, 0)
        nfull = (CHUNK // 16) * 16
        if CHUNK - nfull:
            wv = wb[pl.ds(nfull, 16)]
            for j in range(CHUNK - nfull):
                rb[nfull + j, :] = rb[nfull + j, :] * _lane_splat(wv, j)

    def run_agg():
        def start_gather(k):
            pltpu.async_copy(p_sp.at[src_v[k % 4]], rows_v[k % 2],
                             gsem[k % 2])

        def wait_gather(k):
            pltpu.make_async_copy(p_sp.at[src_v[k % 4]], rows_v[k % 2],
                                  gsem[k % 2]).wait()

        def start_scatter(k):
            pltpu.async_copy(rows_v[k % 2], acc_sp.at[dst_v[k % 4]],
                             ssem[k % 4], add=True)

        def wait_scatter(k):
            pltpu.make_async_copy(rows_v[k % 2], acc_sp.at[dst_v[k % 4]],
                                  ssem[k % 4]).wait()

        start_idx(0, True)
        start_idx(1, True)
        wait_idx(0, True)
        start_gather(0)
        for k in range(NCHUNK):
            wait_gather(k)
            if k >= 1:
                wait_scatter(k - 1)
            if k + 1 < NCHUNK:
                wait_idx(k + 1, True)
                start_gather(k + 1)
            if k + 2 < NCHUNK:
                start_idx(k + 2, True)
            scale_rows(k)
            start_scatter(k)
        wait_scatter(NCHUNK - 1)
        plsc.subcore_barrier()

    run_agg()

    pltpu.sync_copy(acc_sp.at[pl.ds(rbase, ROWS)], a_v)
    b1row = b1_v[...]

    def mid_body(g, carry):
        dv = dinv_v[pl.ds(g * 16, 16)]
        for j in range(16):
            r = g * 16 + j
            dsp = _lane_splat(dv, j)
            s1 = dsp * a_v[r, :] + dsp * dsp * h_v[r, :] + b1row
            rr = jnp.maximum(s1, 0.0)
            h_v[r, :] = rr
            p_v[r, :] = dsp * rr
            a_v[r, :] = jnp.zeros((H,), jnp.float32)
        return carry
    lax.fori_loop(0, ROWS // 16, mid_body, 0)
    pltpu.sync_copy(p_v, p_sp.at[pl.ds(rbase, ROWS)])
    pltpu.sync_copy(a_v, acc_sp.at[pl.ds(rbase, ROWS)])
    plsc.subcore_barrier()

    run_agg()

    pltpu.sync_copy(acc_sp.at[pl.ds(rbase, ROWS)], a_v)

    def out_body(g, carry):
        dv = dinv_v[pl.ds(g * 16, 16)]
        for j in range(16):
            r = g * 16 + j
            dsp = _lane_splat(dv, j)
            a_v[r, :] = dsp * a_v[r, :] + dsp * dsp * h_v[r, :]
        return carry
    lax.fori_loop(0, ROWS // 16, out_body, 0)

    @pl.when((s < 8) == (c == 0))
    def _():
        pltpu.sync_copy(a_v, out_hbm.at[pl.ds(rbase, ROWS)])


def _tc_h1_body(x_ref, w1_ref, h_ref):
    h = jnp.dot(x_ref[:], w1_ref[:], preferred_element_type=jnp.float32)
    h_ref[0:N, :] = h
    h_ref[N:NP, :] = jnp.zeros((NP - N, H), jnp.float32)


def _tc_out_body(s2_ref, w2_ref, b2_ref, o_ref):
    z = jnp.dot(s2_ref[0:N, :], w2_ref[:],
                preferred_element_type=jnp.float32) + b2_ref[:]
    m = jnp.max(z, axis=1, keepdims=True)
    ez = jnp.exp(z - m)
    lse = jnp.log(jnp.sum(ez, axis=1, keepdims=True)) + m
    o_ref[:] = z - lse


_tc_h1 = pl.pallas_call(
    _tc_h1_body,
    out_shape=jax.ShapeDtypeStruct((NP, H), jnp.float32),
)

_tc_out = pl.pallas_call(
    _tc_out_body,
    out_shape=jax.ShapeDtypeStruct((N, C), jnp.float32),
)


@jax.jit
def kernel(x, edge_index, edge_attr, W1, b1, W2, b2):
    src = edge_index[0]
    dst = edge_index[1]
    h1 = _tc_h1(x, W1)
    s2 = _sc_mega(h1, src, dst, edge_attr, b1)
    return _tc_out(s2, W2, b2.reshape(1, C))
